# Initial kernel scaffold; baseline (speedup 1.0000x reference)
import jax
import jax.numpy as jnp
from jax.experimental import pallas as pl


def kernel(node_id_user, node_id_movie, edge_index, emb_user, emb_movie, W1_um, a_src1_um, a_dst1_um, b1_um, W1_mu, a_src1_mu, a_dst1_mu, b1_mu, W2_um, a_src2_um, a_dst2_um, b2_um, W2_mu, a_src2_mu, a_dst2_mu, b2_mu):
    raise NotImplementedError("write your pallas kernel here")


# speedup vs baseline: 1.6322x; 1.6322x over previous
import jax
import jax.numpy as jnp
from jax.experimental import pallas as pl

N_USERS = 25000
N_MOVIES = 25000
D = 128
ROW_BLK = 200


def _finish_body(acc_ref, s_ref, b_ref, o_ref, relu: bool):
    s = s_ref[...]
    o = acc_ref[...] / s + b_ref[...]
    if relu:
        o = jnp.maximum(o, 0.0)
    o_ref[...] = o


def _finish(acc, s, b, relu):
    n = acc.shape[0]
    grid = n // ROW_BLK
    return pl.pallas_call(
        lambda a, sr, br, o: _finish_body(a, sr, br, o, relu),
        grid=(grid,),
        in_specs=[
            pl.BlockSpec((ROW_BLK, D), lambda i: (i, 0)),
            pl.BlockSpec((ROW_BLK, 1), lambda i: (i, 0)),
            pl.BlockSpec((1, D), lambda i: (0, 0)),
        ],
        out_specs=pl.BlockSpec((ROW_BLK, D), lambda i: (i, 0)),
        out_shape=jax.ShapeDtypeStruct((n, D), jnp.float32),
    )(acc, s.reshape(n, 1), b.reshape(1, D))


def _gat_fused(x_src, x_dst, src, dst, n_dst, W, a_s, a_d, b, relu):
    h_src = x_src @ W.T
    al_s = h_src @ a_s
    al_d = x_dst @ (W.T @ a_d)
    e = al_s[src] + al_d[dst]
    e = jnp.maximum(e, 0.2 * e)
    w = jnp.exp(e)
    acc = jax.ops.segment_sum(h_src[src] * w[:, None], dst, num_segments=n_dst)
    s = jax.ops.segment_sum(w, dst, num_segments=n_dst)
    return _finish(acc, s, b, relu)


def kernel(node_id_user, node_id_movie, edge_index, emb_user, emb_movie,
           W1_um, a_src1_um, a_dst1_um, b1_um,
           W1_mu, a_src1_mu, a_dst1_mu, b1_mu,
           W2_um, a_src2_um, a_dst2_um, b2_um,
           W2_mu, a_src2_mu, a_dst2_mu, b2_mu):
    x_u = emb_user
    x_m = emb_movie
    u_idx = edge_index[0]
    m_idx = edge_index[1]
    u1 = _gat_fused(x_m, x_u, m_idx, u_idx, N_USERS, W1_mu, a_src1_mu, a_dst1_mu, b1_mu, True)
    m1 = _gat_fused(x_u, x_m, u_idx, m_idx, N_MOVIES, W1_um, a_src1_um, a_dst1_um, b1_um, True)
    u2 = _gat_fused(m1, u1, m_idx, u_idx, N_USERS, W2_mu, a_src2_mu, a_dst2_mu, b2_mu, False)
    m2 = _gat_fused(u1, m1, u_idx, m_idx, N_MOVIES, W2_um, a_src2_um, a_dst2_um, b2_um, False)
    return jnp.concatenate([u2, m2], axis=0)


# speedup vs baseline: 4.4863x; 2.7486x over previous
import functools

import jax
import jax.numpy as jnp
from jax import lax
from jax.experimental import pallas as pl
from jax.experimental.pallas import tpu as pltpu
from jax.experimental.pallas import tpu_sc as plsc

N = 25000
D = 128
E = 400000

NPADR = 25088
EPAD = 401408
STRIPE_A = EPAD // 32
STRIPE_C = EPAD // 16
NCHUNK_C = STRIPE_C // 128
NPH = 3
PROWS = 8448
TSLICE = PROWS // 16
ROW_BLK = 200

_MESH = plsc.VectorSubcoreMesh(core_axis_name="c", subcore_axis_name="s")
_SC_PARAMS = pltpu.CompilerParams(needs_layout_passes=False,
                                  use_tc_tiling_on_sc=False)
_PAD_SENTINEL = 1 << 20


def _iota16():
    return lax.iota(jnp.int32, 16)


def _full16(v):
    return jnp.full((16,), v, jnp.int32)


def _edge_w_body(als_hbm, ald_hbm, src_hbm, dst_hbm, w_hbm, sp_hbm,
                 als_v, ald_v, src_v, dst_v, w_v, s_vm):
    c = lax.axis_index("c")
    s = lax.axis_index("s")
    wid = s * 2 + c
    base = wid * STRIPE_A
    iota = _iota16()
    zero16 = jnp.zeros((16,), jnp.float32)
    pltpu.sync_copy(als_hbm, als_v)
    pltpu.sync_copy(ald_hbm, ald_v)
    pltpu.sync_copy(src_hbm.at[pl.ds(base, STRIPE_A)], src_v)
    pltpu.sync_copy(dst_hbm.at[pl.ds(base, STRIPE_A)], dst_v)

    def zs(i, carry):
        rf = _full16(i)
        for k in range(8):
            plsc.store_scatter(s_vm, [rf, iota + k * 16], zero16)
        return carry

    lax.fori_loop(0, NPADR // 128, zs, 0)

    def body(i, carry):
        off = i * 16
        s16 = src_v[pl.ds(off, 16)]
        draw = dst_v[pl.ds(off, 16)]
        d16 = jnp.minimum(draw, N - 1)
        e = plsc.load_gather(als_v, [s16]) + plsc.load_gather(ald_v, [d16])
        e = jnp.maximum(e, 0.2 * e)
        w16 = jnp.exp(e)
        w_v[pl.ds(off, 16)] = w16
        wz = jnp.where(draw < N, w16, 0.0)
        plsc.addupdate_scatter(
            s_vm, [lax.shift_right_logical(d16, 7), lax.bitwise_and(d16, 127)],
            wz)
        return carry

    lax.fori_loop(0, STRIPE_A // 16, body, 0)
    pltpu.sync_copy(w_v, w_hbm.at[pl.ds(base, STRIPE_A)])
    pltpu.sync_copy(s_vm, sp_hbm.at[wid])


@functools.partial(
    pl.kernel,
    mesh=_MESH,
    compiler_params=_SC_PARAMS,
    out_type=(
        jax.ShapeDtypeStruct((EPAD,), jnp.float32),
        jax.ShapeDtypeStruct((32, NPADR // 128, 128), jnp.float32),
    ),
    scratch_types=[
        pltpu.VMEM((N,), jnp.float32),
        pltpu.VMEM((N,), jnp.float32),
        pltpu.VMEM((STRIPE_A,), jnp.int32),
        pltpu.VMEM((STRIPE_A,), jnp.int32),
        pltpu.VMEM((STRIPE_A,), jnp.float32),
        pltpu.VMEM((NPADR // 128, 128), jnp.float32),
    ],
)
def _edge_w(*refs):
    _edge_w_body(*refs)


def _scatter_body(h_hbm, src_hbm, dst_hbm, w_hbm, acc_hbm,
                  src_v, dst_v, w_v, rows_v, dl_v, wz_v, acc_sp, sem):
    c = lax.axis_index("c")
    sid = lax.axis_index("s")
    base = sid * STRIPE_C
    iota = _iota16()
    zero16 = jnp.zeros((16,), jnp.float32)

    pltpu.sync_copy(src_hbm.at[pl.ds(base, STRIPE_C)], src_v)
    pltpu.sync_copy(dst_hbm.at[pl.ds(base, STRIPE_C)], dst_v)
    pltpu.sync_copy(w_hbm.at[pl.ds(base, STRIPE_C)], w_v)

    def shift(i, carry):
        off = i * 16
        src_v[pl.ds(off, 16)] = src_v[pl.ds(off, 16)] + c * NPADR
        return carry

    lax.fori_loop(0, STRIPE_C // 16, shift, 0)

    for p in range(NPH):
        lo = p * PROWS

        def zrow(r, carry):
            rf = _full16(r)
            for k in range(4):
                plsc.store_scatter(rows_v, [rf, iota + k * 16], zero16)
            return carry

        lax.fori_loop(0, 128, zrow, 0)

        for j in range(TSLICE // 128):
            pltpu.sync_copy(rows_v,
                            acc_sp.at[pl.ds(sid * TSLICE + j * 128, 128)])
        pltpu.sync_copy(rows_v.at[pl.ds(0, TSLICE % 128)],
                        acc_sp.at[pl.ds(sid * TSLICE + (TSLICE // 128) * 128,
                                        TSLICE % 128)])
        plsc.subcore_barrier()

        def chunk(j, carry):
            cp = pltpu.async_copy(h_hbm.at[src_v.at[pl.ds(j * 128, 128)]],
                                  rows_v, sem)
            for k in range(8):
                off = j * 128 + k * 16
                d16 = dst_v[pl.ds(off, 16)]
                w16 = w_v[pl.ds(off, 16)]
                inp = (d16 >= lo) & (d16 < jnp.minimum(lo + PROWS, N))
                dl = jnp.where(inp, d16 - lo, 0)
                wz = jnp.where(inp, w16, 0.0)
                dl_v[pl.ds(k * 16, 16)] = dl
                wz_v[pl.ds(k * 16, 16)] = wz
            cp.wait()

            def scale(r, inner):
                w16 = plsc.load_gather(wz_v, [_full16(r)])
                rf = _full16(r)
                for k in range(4):
                    col = iota + k * 16
                    v = plsc.load_gather(rows_v, [rf, col])
                    plsc.store_scatter(rows_v, [rf, col], v * w16)
                return inner

            lax.fori_loop(0, 128, scale, 0)
            pltpu.sync_copy(rows_v, acc_sp.at[dl_v], add=True)
            return carry

        lax.fori_loop(0, NCHUNK_C, chunk, 0)
        plsc.subcore_barrier()

        roff = p * 2 * PROWS + c * PROWS + sid * TSLICE
        pltpu.sync_copy(acc_sp.at[pl.ds(sid * TSLICE, TSLICE)],
                        acc_hbm.at[pl.ds(roff, TSLICE)])


@functools.partial(
    pl.kernel,
    mesh=_MESH,
    compiler_params=_SC_PARAMS,
    out_type=jax.ShapeDtypeStruct((NPH * 2 * PROWS, 64), jnp.float32),
    scratch_types=[
        pltpu.VMEM((STRIPE_C,), jnp.int32),
        pltpu.VMEM((STRIPE_C,), jnp.int32),
        pltpu.VMEM((STRIPE_C,), jnp.float32),
        pltpu.VMEM((128, 64), jnp.float32),
        pltpu.VMEM((128,), jnp.int32),
        pltpu.VMEM((128,), jnp.float32),
        pltpu.VMEM_SHARED((PROWS, 64), jnp.float32),
        pltpu.SemaphoreType.DMA,
    ],
)
def _scatter(*refs):
    _scatter_body(*refs)


def _prep_body(x_ref, wo_ref, as_ref, wi_ref, ad_ref, h_ref, als_ref, ald_ref):
    x = x_ref[...]
    h = lax.dot_general(x, wo_ref[...], (((1,), (1,)), ((), ())),
                        preferred_element_type=jnp.float32)
    h_ref[0] = h[:, :64]
    h_ref[1] = h[:, 64:]
    als_ref[...] = jnp.sum(h * as_ref[...], axis=1, keepdims=True)
    vd = lax.dot_general(ad_ref[...], wi_ref[...], (((1,), (0,)), ((), ())),
                         preferred_element_type=jnp.float32)
    ald_ref[...] = jnp.sum(x * vd, axis=1, keepdims=True)


def _prep(x, W_out, a_src, W_in, a_dst):
    n = x.shape[0]
    grid = n // ROW_BLK
    return pl.pallas_call(
        _prep_body,
        grid=(grid,),
        in_specs=[
            pl.BlockSpec((ROW_BLK, D), lambda i: (i, 0)),
            pl.BlockSpec((D, D), lambda i: (0, 0)),
            pl.BlockSpec((1, D), lambda i: (0, 0)),
            pl.BlockSpec((D, D), lambda i: (0, 0)),
            pl.BlockSpec((1, D), lambda i: (0, 0)),
        ],
        out_specs=[
            pl.BlockSpec((2, ROW_BLK, 64), lambda i: (0, i, 0)),
            pl.BlockSpec((ROW_BLK, 1), lambda i: (i, 0)),
            pl.BlockSpec((ROW_BLK, 1), lambda i: (i, 0)),
        ],
        out_shape=[
            jax.ShapeDtypeStruct((2, NPADR, 64), jnp.float32),
            jax.ShapeDtypeStruct((n, 1), jnp.float32),
            jax.ShapeDtypeStruct((n, 1), jnp.float32),
        ],
    )(x, W_out, a_src.reshape(1, D), W_in, a_dst.reshape(1, D))


def _sred_body(p_ref, o_ref):
    o_ref[...] = p_ref[...].sum(axis=0)


def _sreduce(s_parts):
    return pl.pallas_call(
        _sred_body,
        out_shape=jax.ShapeDtypeStruct((NPADR // 128, 128), jnp.float32),
    )(s_parts)


def _finish_body(acc_ref, s_ref, b_ref, o_ref, relu: bool):
    s = jnp.maximum(s_ref[...], 1e-30)
    o = acc_ref[...] / s + b_ref[...]
    if relu:
        o = jnp.maximum(o, 0.0)
    o_ref[...] = o


def _finish(acc, s, b, relu):
    n = acc.shape[0]
    grid = n // ROW_BLK
    return pl.pallas_call(
        lambda a, sr, br, o: _finish_body(a, sr, br, o, relu),
        grid=(grid,),
        in_specs=[
            pl.BlockSpec((ROW_BLK, D), lambda i: (i, 0)),
            pl.BlockSpec((ROW_BLK, 1), lambda i: (i, 0)),
            pl.BlockSpec((1, D), lambda i: (0, 0)),
        ],
        out_specs=pl.BlockSpec((ROW_BLK, D), lambda i: (i, 0)),
        out_shape=jax.ShapeDtypeStruct((n, D), jnp.float32),
    )(acc, s.reshape(n, 1), b.reshape(1, D))


def _pad_dst(idx):
    pad = jnp.full((EPAD - E,), _PAD_SENTINEL, jnp.int32)
    return jnp.concatenate([idx, pad])


def _pad_src(idx):
    return jnp.concatenate([idx, jnp.zeros((EPAD - E,), jnp.int32)])


def _pass(h2, al_s, al_d, srcp, dstp, b, relu):
    w, s_parts = _edge_w(al_s.reshape(-1), al_d.reshape(-1), srcp, dstp)
    acc4 = _scatter(h2.reshape(2 * NPADR, 64), srcp, dstp, w)
    a = acc4.reshape(NPH, 2, PROWS, 64)
    parts = []
    for p in range(NPH):
        valid = min(PROWS, N - p * PROWS)
        parts.append(jnp.concatenate([a[p, 0, :valid], a[p, 1, :valid]],
                                     axis=1))
    acc = jnp.concatenate(parts, axis=0)
    s = _sreduce(s_parts).reshape(NPADR)[:N]
    return _finish(acc, s, b, relu)


def kernel(node_id_user, node_id_movie, edge_index, emb_user, emb_movie,
           W1_um, a_src1_um, a_dst1_um, b1_um,
           W1_mu, a_src1_mu, a_dst1_mu, b1_mu,
           W2_um, a_src2_um, a_dst2_um, b2_um,
           W2_mu, a_src2_mu, a_dst2_mu, b2_mu):
    x_u = emb_user
    x_m = emb_movie
    u_idx = edge_index[0]
    m_idx = edge_index[1]

    u_src = _pad_src(u_idx)
    m_src = _pad_src(m_idx)
    u_dst = _pad_dst(u_idx)
    m_dst = _pad_dst(m_idx)

    h_u1, als_um1, ald_mu1 = _prep(x_u, W1_um, a_src1_um, W1_mu, a_dst1_mu)
    h_m1, als_mu1, ald_um1 = _prep(x_m, W1_mu, a_src1_mu, W1_um, a_dst1_um)
    u1 = _pass(h_m1, als_mu1, ald_mu1, m_src, u_dst, b1_mu, True)
    m1 = _pass(h_u1, als_um1, ald_um1, u_src, m_dst, b1_um, True)

    h_u2, als_um2, ald_mu2 = _prep(u1, W2_um, a_src2_um, W2_mu, a_dst2_mu)
    h_m2, als_mu2, ald_um2 = _prep(m1, W2_mu, a_src2_mu, W2_um, a_dst2_um)
    u2 = _pass(h_m2, als_mu2, ald_mu2, m_src, u_dst, b2_mu, False)
    m2 = _pass(h_u2, als_um2, ald_um2, u_src, m_dst, b2_um, False)

    return jnp.concatenate([u2, m2], axis=0)


# speedup vs baseline: 9.0490x; 2.0170x over previous
import functools

import jax
import jax.numpy as jnp
from jax import lax
from jax.experimental import pallas as pl
from jax.experimental.pallas import tpu as pltpu
from jax.experimental.pallas import tpu_sc as plsc

N = 25000
D = 128
E = 400000

NPADR = 25088
EPAD = 401408
STRIPE_A = EPAD // 32
STRIPE_C = EPAD // 16
NCHUNK_C = STRIPE_C // 128
NPH = 3
PROWS = 8448
TSLICE = PROWS // 16
ROW_BLK = 200

_MESH = plsc.VectorSubcoreMesh(core_axis_name="c", subcore_axis_name="s")
_SC_PARAMS = pltpu.CompilerParams(needs_layout_passes=False,
                                  use_tc_tiling_on_sc=False)
_PAD_SENTINEL = 1 << 20


def _iota16():
    return lax.iota(jnp.int32, 16)


def _full16(v):
    return jnp.full((16,), v, jnp.int32)


def _edge_w_body(als_hbm, ald_hbm, src_hbm, dst_hbm, w_hbm, sp_hbm,
                 als_v, ald_v, src_v, dst_v, w_v, s_vm):
    c = lax.axis_index("c")
    s = lax.axis_index("s")
    wid = s * 2 + c
    base = wid * STRIPE_A
    iota = _iota16()
    zero16 = jnp.zeros((16,), jnp.float32)
    pltpu.sync_copy(als_hbm, als_v)
    pltpu.sync_copy(ald_hbm, ald_v)
    pltpu.sync_copy(src_hbm.at[pl.ds(base, STRIPE_A)], src_v)
    pltpu.sync_copy(dst_hbm.at[pl.ds(base, STRIPE_A)], dst_v)

    def zs(i, carry):
        rf = _full16(i)
        for k in range(8):
            plsc.store_scatter(s_vm, [rf, iota + k * 16], zero16)
        return carry

    lax.fori_loop(0, NPADR // 128, zs, 0)

    def body(i, carry):
        off = i * 16
        s16 = src_v[pl.ds(off, 16)]
        draw = dst_v[pl.ds(off, 16)]
        d16 = jnp.minimum(draw, N - 1)
        e = plsc.load_gather(als_v, [s16]) + plsc.load_gather(ald_v, [d16])
        e = jnp.maximum(e, 0.2 * e)
        w16 = jnp.exp(e)
        w_v[pl.ds(off, 16)] = w16
        wz = jnp.where(draw < N, w16, 0.0)
        plsc.addupdate_scatter(
            s_vm, [lax.shift_right_logical(d16, 7), lax.bitwise_and(d16, 127)],
            wz)
        return carry

    lax.fori_loop(0, STRIPE_A // 16, body, 0)
    pltpu.sync_copy(w_v, w_hbm.at[pl.ds(base, STRIPE_A)])
    pltpu.sync_copy(s_vm, sp_hbm.at[wid])


@functools.partial(
    pl.kernel,
    mesh=_MESH,
    compiler_params=_SC_PARAMS,
    out_type=(
        jax.ShapeDtypeStruct((EPAD,), jnp.float32),
        jax.ShapeDtypeStruct((32, NPADR // 128, 128), jnp.float32),
    ),
    scratch_types=[
        pltpu.VMEM((N,), jnp.float32),
        pltpu.VMEM((N,), jnp.float32),
        pltpu.VMEM((STRIPE_A,), jnp.int32),
        pltpu.VMEM((STRIPE_A,), jnp.int32),
        pltpu.VMEM((STRIPE_A,), jnp.float32),
        pltpu.VMEM((NPADR // 128, 128), jnp.float32),
    ],
)
def _edge_w(*refs):
    _edge_w_body(*refs)


def _scatter_body(h_hbm, src_hbm, dst_hbm, w_hbm, acc_hbm,
                  src_v, dst_v, w_v, rows_a, rows_b,
                  dl_a, wz_a, dl_b, wz_b, acc_sp, sem_a, sem_b):
    c = lax.axis_index("c")
    sid = lax.axis_index("s")
    base = sid * STRIPE_C
    iota = _iota16()
    zero16 = jnp.zeros((16,), jnp.float32)

    pltpu.sync_copy(src_hbm.at[pl.ds(base, STRIPE_C)], src_v)
    pltpu.sync_copy(dst_hbm.at[pl.ds(base, STRIPE_C)], dst_v)
    pltpu.sync_copy(w_hbm.at[pl.ds(base, STRIPE_C)], w_v)

    def shift(i, carry):
        off = i * 16
        src_v[pl.ds(off, 16)] = src_v[pl.ds(off, 16)] + c * NPADR
        return carry

    lax.fori_loop(0, STRIPE_C // 16, shift, 0)

    def _gather(j, rows, sem):
        return pltpu.async_copy(h_hbm.at[src_v.at[pl.ds(j * 128, 128)]],
                                rows, sem)

    def _mask(j, lo, dl_v, wz_v):
        for k in range(8):
            off = j * 128 + k * 16
            d16 = dst_v[pl.ds(off, 16)]
            w16 = w_v[pl.ds(off, 16)]
            inp = (d16 >= lo) & (d16 < jnp.minimum(lo + PROWS, N))
            dl_v[pl.ds(k * 16, 16)] = jnp.where(inp, d16 - lo, 0)
            wz_v[pl.ds(k * 16, 16)] = jnp.where(inp, w16, 0.0)

    def _scale(rows, wz_v):
        @functools.partial(plsc.parallel_loop, 0, 128, unroll=8)
        def _(r):
            w16 = plsc.load_gather(wz_v, [_full16(r)])
            for k in range(4):
                sl = pl.ds(k * 16, 16)
                rows[r, sl] = rows[r, sl] * w16

    for p in range(NPH):
        lo = p * PROWS

        def zrow(r, carry):
            rf = _full16(r)
            for k in range(4):
                plsc.store_scatter(rows_a, [rf, iota + k * 16], zero16)
            return carry

        lax.fori_loop(0, 128, zrow, 0)

        for j in range(TSLICE // 128):
            pltpu.sync_copy(rows_a,
                            acc_sp.at[pl.ds(sid * TSLICE + j * 128, 128)])
        pltpu.sync_copy(rows_a.at[pl.ds(0, TSLICE % 128)],
                        acc_sp.at[pl.ds(sid * TSLICE + (TSLICE // 128) * 128,
                                        TSLICE % 128)])
        plsc.subcore_barrier()

        _gather(0, rows_a, sem_a)

        def chunk2(i, carry):
            j0 = 2 * i
            j1 = 2 * i + 1
            _gather(j1, rows_b, sem_b)
            _mask(j0, lo, dl_a, wz_a)
            pltpu.make_async_copy(
                h_hbm.at[src_v.at[pl.ds(j0 * 128, 128)]], rows_a, sem_a
            ).wait()
            _scale(rows_a, wz_a)
            pltpu.sync_copy(rows_a, acc_sp.at[dl_a], add=True)
            _gather(jnp.minimum(j0 + 2, NCHUNK_C - 1), rows_a, sem_a)
            _mask(j1, lo, dl_b, wz_b)
            pltpu.make_async_copy(
                h_hbm.at[src_v.at[pl.ds(j1 * 128, 128)]], rows_b, sem_b
            ).wait()
            _scale(rows_b, wz_b)
            pltpu.sync_copy(rows_b, acc_sp.at[dl_b], add=True)
            return carry

        lax.fori_loop(0, NCHUNK_C // 2, chunk2, 0)
        pltpu.make_async_copy(
            h_hbm.at[src_v.at[pl.ds((NCHUNK_C - 1) * 128, 128)]],
            rows_a, sem_a).wait()
        plsc.subcore_barrier()

        roff = p * 2 * PROWS + c * PROWS + sid * TSLICE
        pltpu.sync_copy(acc_sp.at[pl.ds(sid * TSLICE, TSLICE)],
                        acc_hbm.at[pl.ds(roff, TSLICE)])


@functools.partial(
    pl.kernel,
    mesh=_MESH,
    compiler_params=_SC_PARAMS,
    out_type=jax.ShapeDtypeStruct((NPH * 2 * PROWS, 64), jnp.float32),
    scratch_types=[
        pltpu.VMEM((STRIPE_C,), jnp.int32),
        pltpu.VMEM((STRIPE_C,), jnp.int32),
        pltpu.VMEM((STRIPE_C,), jnp.float32),
        pltpu.VMEM((128, 64), jnp.float32),
        pltpu.VMEM((128, 64), jnp.float32),
        pltpu.VMEM((128,), jnp.int32),
        pltpu.VMEM((128,), jnp.float32),
        pltpu.VMEM((128,), jnp.int32),
        pltpu.VMEM((128,), jnp.float32),
        pltpu.VMEM_SHARED((PROWS, 64), jnp.float32),
        pltpu.SemaphoreType.DMA,
        pltpu.SemaphoreType.DMA,
    ],
)
def _scatter(*refs):
    _scatter_body(*refs)


def _prep_body(x_ref, wo_ref, as_ref, wi_ref, ad_ref, h_ref, als_ref, ald_ref):
    x = x_ref[...]
    h = lax.dot_general(x, wo_ref[...], (((1,), (1,)), ((), ())),
                        preferred_element_type=jnp.float32)
    h_ref[0] = h[:, :64]
    h_ref[1] = h[:, 64:]
    als_ref[...] = jnp.sum(h * as_ref[...], axis=1, keepdims=True)
    vd = lax.dot_general(ad_ref[...], wi_ref[...], (((1,), (0,)), ((), ())),
                         preferred_element_type=jnp.float32)
    ald_ref[...] = jnp.sum(x * vd, axis=1, keepdims=True)


def _prep(x, W_out, a_src, W_in, a_dst):
    n = x.shape[0]
    grid = n // ROW_BLK
    return pl.pallas_call(
        _prep_body,
        grid=(grid,),
        in_specs=[
            pl.BlockSpec((ROW_BLK, D), lambda i: (i, 0)),
            pl.BlockSpec((D, D), lambda i: (0, 0)),
            pl.BlockSpec((1, D), lambda i: (0, 0)),
            pl.BlockSpec((D, D), lambda i: (0, 0)),
            pl.BlockSpec((1, D), lambda i: (0, 0)),
        ],
        out_specs=[
            pl.BlockSpec((2, ROW_BLK, 64), lambda i: (0, i, 0)),
            pl.BlockSpec((ROW_BLK, 1), lambda i: (i, 0)),
            pl.BlockSpec((ROW_BLK, 1), lambda i: (i, 0)),
        ],
        out_shape=[
            jax.ShapeDtypeStruct((2, NPADR, 64), jnp.float32),
            jax.ShapeDtypeStruct((n, 1), jnp.float32),
            jax.ShapeDtypeStruct((n, 1), jnp.float32),
        ],
    )(x, W_out, a_src.reshape(1, D), W_in, a_dst.reshape(1, D))


def _sred_body(p_ref, o_ref):
    o_ref[...] = p_ref[...].sum(axis=0)


def _sreduce(s_parts):
    return pl.pallas_call(
        _sred_body,
        out_shape=jax.ShapeDtypeStruct((NPADR // 128, 128), jnp.float32),
    )(s_parts)


def _finish_body(acc_ref, s_ref, b_ref, o_ref, relu: bool):
    s = jnp.maximum(s_ref[...], 1e-30)
    o = acc_ref[...] / s + b_ref[...]
    if relu:
        o = jnp.maximum(o, 0.0)
    o_ref[...] = o


def _finish(acc, s, b, relu):
    n = acc.shape[0]
    grid = n // ROW_BLK
    return pl.pallas_call(
        lambda a, sr, br, o: _finish_body(a, sr, br, o, relu),
        grid=(grid,),
        in_specs=[
            pl.BlockSpec((ROW_BLK, D), lambda i: (i, 0)),
            pl.BlockSpec((ROW_BLK, 1), lambda i: (i, 0)),
            pl.BlockSpec((1, D), lambda i: (0, 0)),
        ],
        out_specs=pl.BlockSpec((ROW_BLK, D), lambda i: (i, 0)),
        out_shape=jax.ShapeDtypeStruct((n, D), jnp.float32),
    )(acc, s.reshape(n, 1), b.reshape(1, D))


def _pad_dst(idx):
    pad = jnp.full((EPAD - E,), _PAD_SENTINEL, jnp.int32)
    return jnp.concatenate([idx, pad])


def _pad_src(idx):
    return jnp.concatenate([idx, jnp.zeros((EPAD - E,), jnp.int32)])


def _pass(h2, al_s, al_d, srcp, dstp, b, relu):
    w, s_parts = _edge_w(al_s.reshape(-1), al_d.reshape(-1), srcp, dstp)
    acc4 = _scatter(h2.reshape(2 * NPADR, 64), srcp, dstp, w)
    a = acc4.reshape(NPH, 2, PROWS, 64)
    parts = []
    for p in range(NPH):
        valid = min(PROWS, N - p * PROWS)
        parts.append(jnp.concatenate([a[p, 0, :valid], a[p, 1, :valid]],
                                     axis=1))
    acc = jnp.concatenate(parts, axis=0)
    s = _sreduce(s_parts).reshape(NPADR)[:N]
    return _finish(acc, s, b, relu)


def kernel(node_id_user, node_id_movie, edge_index, emb_user, emb_movie,
           W1_um, a_src1_um, a_dst1_um, b1_um,
           W1_mu, a_src1_mu, a_dst1_mu, b1_mu,
           W2_um, a_src2_um, a_dst2_um, b2_um,
           W2_mu, a_src2_mu, a_dst2_mu, b2_mu):
    x_u = emb_user
    x_m = emb_movie
    u_idx = edge_index[0]
    m_idx = edge_index[1]

    u_src = _pad_src(u_idx)
    m_src = _pad_src(m_idx)
    u_dst = _pad_dst(u_idx)
    m_dst = _pad_dst(m_idx)

    h_u1, als_um1, ald_mu1 = _prep(x_u, W1_um, a_src1_um, W1_mu, a_dst1_mu)
    h_m1, als_mu1, ald_um1 = _prep(x_m, W1_mu, a_src1_mu, W1_um, a_dst1_um)
    u1 = _pass(h_m1, als_mu1, ald_mu1, m_src, u_dst, b1_mu, True)
    m1 = _pass(h_u1, als_um1, ald_um1, u_src, m_dst, b1_um, True)

    h_u2, als_um2, ald_mu2 = _prep(u1, W2_um, a_src2_um, W2_mu, a_dst2_mu)
    h_m2, als_mu2, ald_um2 = _prep(m1, W2_mu, a_src2_mu, W2_um, a_dst2_um)
    u2 = _pass(h_m2, als_mu2, ald_mu2, m_src, u_dst, b2_mu, False)
    m2 = _pass(h_u2, als_um2, ald_um2, u_src, m_dst, b2_um, False)

    return jnp.concatenate([u2, m2], axis=0)
